# Initial kernel scaffold; baseline (speedup 1.0000x reference)
import jax
import jax.numpy as jnp
from jax.experimental import pallas as pl


def kernel(O, tables):
    raise NotImplementedError("write your pallas kernel here")


# speedup vs baseline: 1.1305x; 1.1305x over previous
import functools

import jax
import jax.numpy as jnp
from jax import lax
from jax.experimental import pallas as pl
from jax.experimental.pallas import tpu as pltpu
from jax.experimental.pallas import tpu_sc as plsc

_NUM = 13
_NCAT = 26
_VOCAB = 100000
_D = 32
_B = 16384
_F = _NUM + _NCAT * _D

_NC, _NS, _L = 2, 16, 16
_NW = _NC * _NS
_BPW = _B // _NW
_R = 64
_NSUB = _BPW // _R

_mesh = plsc.VectorSubcoreMesh(
    core_axis_name="c", subcore_axis_name="s", num_cores=_NC, num_subcores=_NS
)


@functools.partial(
    pl.kernel,
    out_type=jax.ShapeDtypeStruct((_B, _F), jnp.float32),
    mesh=_mesh,
    scratch_types=[
        pltpu.VMEM((_NCAT, _BPW), jnp.int32),
        pltpu.VMEM((_NCAT, _R, _D), jnp.float32),
        pltpu.VMEM((_R, _F), jnp.float32),
        pltpu.VMEM((_R, _L), jnp.int32),
        pltpu.SemaphoreType.DMA,
        pltpu.SemaphoreType.DMA,
        pltpu.SemaphoreType.DMA,
    ],
    compiler_params=pltpu.CompilerParams(use_tc_tiling_on_sc=False),
)
def _sc_embed(o_hbm, ot_hbm, tab_hbm, out_hbm, idx_v, stage, buf, ni_v,
              sem_i, sem_n, sem_g):
    wid = lax.axis_index("s") * _NC + lax.axis_index("c")
    base = wid * _BPW

    idx_cps = [
        pltpu.async_copy(
            ot_hbm.at[_NUM + i, pl.ds(base, _BPW)], idx_v.at[i], sem_i
        )
        for i in range(_NCAT)
    ]
    for cp in idx_cps:
        cp.wait()

    def addoff(t, c):
        i = t // (_BPW // _L)
        k = t % (_BPW // _L)
        idx_v[i, pl.ds(k * _L, _L)] = idx_v[i, pl.ds(k * _L, _L)] + i * _VOCAB
        return c

    lax.fori_loop(0, _NCAT * (_BPW // _L), addoff, 0)

    def sub(s, c):
        sbase = base + s * _R
        n_cp = pltpu.async_copy(
            o_hbm.at[pl.ds(sbase, _R), pl.ds(0, _L)], ni_v, sem_n
        )
        g_cps = [
            pltpu.async_copy(
                tab_hbm.at[idx_v.at[i, pl.ds(s * _R, _R)]], stage.at[i], sem_g
            )
            for i in range(_NCAT)
        ]
        n_cp.wait()
        for cp in g_cps:
            cp.wait()

        def asm(r, c2):
            buf[r, pl.ds(0, _L)] = ni_v[r].astype(jnp.float32) * (1.0 / _VOCAB)
            for i in range(_NCAT):
                col = _NUM + _D * i
                buf[r, pl.ds(col, _L)] = stage[i, r, pl.ds(0, _L)]
                buf[r, pl.ds(col + _L, _L)] = stage[i, r, pl.ds(_L, _L)]
            return c2

        lax.fori_loop(0, _R, asm, 0)

        pltpu.sync_copy(buf, out_hbm.at[pl.ds(sbase, _R)])
        return c

    lax.fori_loop(0, _NSUB, sub, 0)


def kernel(O, tables):
    O_T = O.T
    tab_flat = tables.reshape(_NCAT * _VOCAB, _D)
    return _sc_embed(O, O_T, tab_flat)


# speedup vs baseline: 1.1334x; 1.0025x over previous
import functools

import jax
import jax.numpy as jnp
from jax import lax
from jax.experimental import pallas as pl
from jax.experimental.pallas import tpu as pltpu
from jax.experimental.pallas import tpu_sc as plsc

_NUM = 13
_NCAT = 26
_VOCAB = 100000
_D = 32
_B = 16384
_F = _NUM + _NCAT * _D

_NC, _NS, _L = 2, 16, 16
_NW = _NC * _NS
_BPW = _B // _NW
_R = 64
_NSUB = _BPW // _R

_mesh = plsc.VectorSubcoreMesh(
    core_axis_name="c", subcore_axis_name="s", num_cores=_NC, num_subcores=_NS
)


@functools.partial(
    pl.kernel,
    out_type=jax.ShapeDtypeStruct((_B, _F), jnp.float32),
    mesh=_mesh,
    scratch_types=[
        pltpu.VMEM((_NCAT, _BPW), jnp.int32),
        pltpu.VMEM((_NCAT, _R, _D), jnp.float32),
        pltpu.VMEM((_R, _F), jnp.float32),
        pltpu.VMEM((_R, _L), jnp.int32),
        pltpu.SemaphoreType.DMA,
        pltpu.SemaphoreType.DMA,
        pltpu.SemaphoreType.DMA,
        pltpu.SemaphoreType.DMA,
    ],
    compiler_params=pltpu.CompilerParams(use_tc_tiling_on_sc=False),
)
def _sc_embed(o_hbm, cat_hbm, tab_hbm, out_hbm, idx_v, stage, buf, ni_v,
              sem_i, sem_n, sem_g, sem_w):
    wid = lax.axis_index("s") * _NC + lax.axis_index("c")
    base = wid * _BPW

    idx_cps = [
        pltpu.async_copy(
            cat_hbm.at[pl.ds(i * _B + base, _BPW)], idx_v.at[i], sem_i
        )
        for i in range(_NCAT)
    ]
    for cp in idx_cps:
        cp.wait()

    def addoff(t, c):
        i = t // (_BPW // _L)
        k = t % (_BPW // _L)
        idx_v[i, pl.ds(k * _L, _L)] = idx_v[i, pl.ds(k * _L, _L)] + i * _VOCAB
        return c

    lax.fori_loop(0, _NCAT * (_BPW // _L), addoff, 0)

    def sub(s, c):
        sbase = base + s * _R
        n_cp = pltpu.async_copy(
            o_hbm.at[pl.ds(sbase, _R), pl.ds(0, _L)], ni_v, sem_n
        )
        g_cps = [
            pltpu.async_copy(
                tab_hbm.at[idx_v.at[i, pl.ds(s * _R, _R)]], stage.at[i], sem_g
            )
            for i in range(_NCAT)
        ]
        n_cp.wait()
        for cp in g_cps:
            cp.wait()

        w_drain = pltpu.make_async_copy(buf, out_hbm.at[pl.ds(sbase, _R)], sem_w)

        @pl.when(s > 0)
        def _():
            w_drain.wait()

        def asm(r, c2):
            buf[r, pl.ds(0, _L)] = ni_v[r].astype(jnp.float32) * (1.0 / _VOCAB)
            for i in range(_NCAT):
                col = _NUM + _D * i
                buf[r, pl.ds(col, _L)] = stage[i, r, pl.ds(0, _L)]
                buf[r, pl.ds(col + _L, _L)] = stage[i, r, pl.ds(_L, _L)]
            return c2

        lax.fori_loop(0, _R, asm, 0)

        pltpu.async_copy(buf, out_hbm.at[pl.ds(sbase, _R)], sem_w)
        return c

    lax.fori_loop(0, _NSUB, sub, 0)
    pltpu.make_async_copy(buf, out_hbm.at[pl.ds(base, _R)], sem_w).wait()


def kernel(O, tables):
    cat_flat = O[:, _NUM:].T.reshape(_NCAT * _B)
    tab_flat = tables.reshape(_NCAT * _VOCAB, _D)
    return _sc_embed(O, cat_flat, tab_flat)


# speedup vs baseline: 1.5364x; 1.3556x over previous
import functools

import jax
import jax.numpy as jnp
from jax import lax
from jax.experimental import pallas as pl
from jax.experimental.pallas import tpu as pltpu
from jax.experimental.pallas import tpu_sc as plsc

_NUM = 13
_NCAT = 26
_VOCAB = 100000
_D = 32
_B = 16384
_F = _NUM + _NCAT * _D

_NC, _NS, _L = 2, 16, 16
_NW = _NC * _NS
_BPW = _B // _NW
_R = 64
_NSUB = _BPW // _R

_mesh = plsc.VectorSubcoreMesh(
    core_axis_name="c", subcore_axis_name="s", num_cores=_NC, num_subcores=_NS
)


@functools.partial(
    pl.kernel,
    out_type=jax.ShapeDtypeStruct((_B, _F), jnp.float32),
    mesh=_mesh,
    scratch_types=[
        pltpu.VMEM((_NCAT, _BPW), jnp.int32),
        pltpu.VMEM((_NCAT, _R, _D), jnp.float32),
        pltpu.VMEM((_R, _F), jnp.float32),
        pltpu.VMEM((_R, _L), jnp.int32),
        pltpu.SemaphoreType.DMA,
        pltpu.SemaphoreType.DMA,
        pltpu.SemaphoreType.DMA,
        pltpu.SemaphoreType.DMA,
    ],
    compiler_params=pltpu.CompilerParams(use_tc_tiling_on_sc=False),
)
def _sc_embed(o_hbm, cat_hbm, tab_hbm, out_hbm, idx_v, stage, buf, ni_v,
              sem_i, sem_n, sem_g, sem_w):
    wid = lax.axis_index("s") * _NC + lax.axis_index("c")
    base = wid * _BPW

    idx_cps = [
        pltpu.async_copy(
            cat_hbm.at[pl.ds(i * _B + base, _BPW)], idx_v.at[i], sem_i
        )
        for i in range(_NCAT)
    ]
    for cp in idx_cps:
        cp.wait()

    def addoff(t, c):
        i = t // (_BPW // _L)
        k = t % (_BPW // _L)
        v = idx_v[i, pl.ds(k * _L, _L)]
        blk = (v >> 12) << 12
        j = (v & (_VB - 1)) >> 10
        r = v & (_Q - 1)
        idx_v[i, pl.ds(k * _L, _L)] = i * _VP + blk + (r << 2) + j
        return c

    lax.fori_loop(0, _NCAT * (_BPW // _L), addoff, 0)

    def sub(s, c):
        sbase = base + s * _R
        n_cp = pltpu.async_copy(
            o_hbm.at[pl.ds(sbase, _R), pl.ds(0, _L)], ni_v, sem_n
        )
        g_cps = [
            pltpu.async_copy(
                tab_hbm.at[idx_v.at[i, pl.ds(s * _R, _R)]], stage.at[i], sem_g
            )
            for i in range(_NCAT)
        ]
        n_cp.wait()
        for cp in g_cps:
            cp.wait()

        w_drain = pltpu.make_async_copy(buf, out_hbm.at[pl.ds(sbase, _R)], sem_w)

        @pl.when(s > 0)
        def _():
            w_drain.wait()

        def asm(r, c2):
            buf[r, pl.ds(0, _L)] = ni_v[r].astype(jnp.float32) * (1.0 / _VOCAB)
            for i in range(_NCAT):
                col = _NUM + _D * i
                buf[r, pl.ds(col, _L)] = stage[i, r, pl.ds(0, _L)]
                buf[r, pl.ds(col + _L, _L)] = stage[i, r, pl.ds(_L, _L)]
            return c2

        lax.fori_loop(0, _R, asm, 0)

        pltpu.async_copy(buf, out_hbm.at[pl.ds(sbase, _R)], sem_w)
        return c

    lax.fori_loop(0, _NSUB, sub, 0)
    pltpu.make_async_copy(buf, out_hbm.at[pl.ds(base, _R)], sem_w).wait()


_VP = 102400
_VB = 4096
_NVB = _VP // _VB
_Q = _VB // 4


def _detile_body(in_ref, out_ref):
    x = in_ref[0]
    y = x.T
    for j in range(4):
        out_ref[:, pl.ds(j * _D, _D)] = y[j * (_VB // 4):(j + 1) * (_VB // 4)]


_tc_detile = pl.pallas_call(
    _detile_body,
    grid=(_NCAT, _NVB),
    in_specs=[pl.BlockSpec((1, _D, _VB), lambda i, b: (i, 0, b))],
    out_specs=pl.BlockSpec(
        (_VB * _D // 128, 128), lambda i, b: (i * _NVB + b, 0)
    ),
    out_shape=jax.ShapeDtypeStruct((_NCAT * _VP * _D // 128, 128), jnp.float32),
)


def kernel(O, tables):
    cat_flat = O[:, _NUM:].T.reshape(_NCAT * _B)
    tab128 = _tc_detile(tables.transpose(0, 2, 1))
    tab_flat = tab128.reshape(_NCAT * _VP, _D)
    return _sc_embed(O, cat_flat, tab_flat)


# speedup vs baseline: 2.8748x; 1.8712x over previous
import functools

import jax
import jax.numpy as jnp
from jax import lax
from jax.experimental import pallas as pl
from jax.experimental.pallas import tpu as pltpu
from jax.experimental.pallas import tpu_sc as plsc

_NUM = 13
_NCAT = 26
_VOCAB = 100000
_D = 32
_B = 16384
_F = _NUM + _NCAT * _D

_NC, _NS, _L = 2, 16, 16
_NW = _NC * _NS
_BPW = _B // _NW
_R = 64
_NSUB = _BPW // _R

_mesh = plsc.VectorSubcoreMesh(
    core_axis_name="c", subcore_axis_name="s", num_cores=_NC, num_subcores=_NS
)


@functools.partial(
    pl.kernel,
    out_type=jax.ShapeDtypeStruct((_B, _F), jnp.float32),
    mesh=_mesh,
    scratch_types=[
        pltpu.VMEM((_NCAT, _BPW), jnp.int32),
        pltpu.VMEM((_NCAT, _R, _D), jnp.float32),
        pltpu.VMEM((_R, _F), jnp.float32),
        pltpu.VMEM((_R, _L), jnp.int32),
        pltpu.SemaphoreType.DMA,
        pltpu.SemaphoreType.DMA,
        pltpu.SemaphoreType.DMA,
        pltpu.SemaphoreType.DMA,
    ],
    compiler_params=pltpu.CompilerParams(use_tc_tiling_on_sc=False),
)
def _sc_embed(o_hbm, cat_hbm, tab_hbm, out_hbm, idx_v, stage, buf, ni_v,
              sem_i, sem_n, sem_g, sem_w):
    wid = lax.axis_index("s") * _NC + lax.axis_index("c")
    base = wid * _BPW

    idx_cps = [
        pltpu.async_copy(
            cat_hbm.at[pl.ds(i * _B + base, _BPW)], idx_v.at[i], sem_i
        )
        for i in range(_NCAT)
    ]
    for cp in idx_cps:
        cp.wait()

    def addoff(t, c):
        i = t // (_BPW // _L)
        k = t % (_BPW // _L)
        v = idx_v[i, pl.ds(k * _L, _L)]
        gbase = (i >> 2) * (4 * _VP) + (i & 3)
        idx_v[i, pl.ds(k * _L, _L)] = gbase + (v << 2)
        return c

    lax.fori_loop(0, _NCAT * (_BPW // _L), addoff, 0)

    def sub(s, c):
        sbase = base + s * _R
        n_cp = pltpu.async_copy(
            o_hbm.at[pl.ds(sbase, _R), pl.ds(0, _L)], ni_v, sem_n
        )
        g_cps = [
            pltpu.async_copy(
                tab_hbm.at[idx_v.at[i, pl.ds(s * _R, _R)]], stage.at[i], sem_g
            )
            for i in range(_NCAT)
        ]
        n_cp.wait()
        for cp in g_cps:
            cp.wait()

        w_drain = pltpu.make_async_copy(buf, out_hbm.at[pl.ds(sbase, _R)], sem_w)

        @pl.when(s > 0)
        def _():
            w_drain.wait()

        def asm(r, c2):
            buf[r, pl.ds(0, _L)] = ni_v[r].astype(jnp.float32) * (1.0 / _VOCAB)
            for i in range(_NCAT):
                col = _NUM + _D * i
                buf[r, pl.ds(col, _L)] = stage[i, r, pl.ds(0, _L)]
                buf[r, pl.ds(col + _L, _L)] = stage[i, r, pl.ds(_L, _L)]
            return c2

        lax.fori_loop(0, _R, asm, 0)

        pltpu.async_copy(buf, out_hbm.at[pl.ds(sbase, _R)], sem_w)
        return c

    lax.fori_loop(0, _NSUB, sub, 0)
    pltpu.make_async_copy(buf, out_hbm.at[pl.ds(base, _R)], sem_w).wait()


_VP = 102400
_VB = 4096
_NVB = _VP // _VB
_NG = 7


def _detile_body(in_ref, out_ref):
    x = in_ref[...]
    x128 = x.reshape(4 * _D, _VB)
    out_ref[...] = x128.T


_tc_detile = pl.pallas_call(
    _detile_body,
    grid=(_NG, _NVB),
    in_specs=[pl.BlockSpec((4, _D, _VB), lambda g, b: (g, 0, b))],
    out_specs=pl.BlockSpec((_VB, 128), lambda g, b: (g * _NVB + b, 0)),
    out_shape=jax.ShapeDtypeStruct((_NG * _VP, 128), jnp.float32),
)


def kernel(O, tables):
    cat_flat = O[:, _NUM:].T.reshape(_NCAT * _B)
    tab128 = _tc_detile(tables.transpose(0, 2, 1))
    tab_flat = tab128.reshape(_NG * _VP * 4, _D)
    return _sc_embed(O, cat_flat, tab_flat)


# speedup vs baseline: 2.9541x; 1.0276x over previous
import functools

import jax
import jax.numpy as jnp
from jax import lax
from jax.experimental import pallas as pl
from jax.experimental.pallas import tpu as pltpu
from jax.experimental.pallas import tpu_sc as plsc

_NUM = 13
_NCAT = 26
_VOCAB = 100000
_D = 32
_B = 16384
_F = _NUM + _NCAT * _D

_NC, _NS, _L = 2, 16, 16
_NW = _NC * _NS
_BPW = _B // _NW
_R = 64
_NSUB = _BPW // _R

_mesh = plsc.VectorSubcoreMesh(
    core_axis_name="c", subcore_axis_name="s", num_cores=_NC, num_subcores=_NS
)


@functools.partial(
    pl.kernel,
    out_type=jax.ShapeDtypeStruct((_B, _F), jnp.float32),
    mesh=_mesh,
    scratch_types=[
        pltpu.VMEM((_NCAT, _BPW), jnp.int32),
        pltpu.VMEM((_NCAT, _R, _D), jnp.float32),
        pltpu.VMEM((_R, _F), jnp.float32),
        pltpu.VMEM((_R, _L), jnp.int32),
        pltpu.SemaphoreType.DMA,
        pltpu.SemaphoreType.DMA,
        pltpu.SemaphoreType.DMA,
        pltpu.SemaphoreType.DMA,
    ],
    compiler_params=pltpu.CompilerParams(use_tc_tiling_on_sc=False),
)
def _sc_embed(o_hbm, cat_hbm, tab_hbm, out_hbm, idx_v, stage, buf, ni_v,
              sem_i, sem_n, sem_g, sem_w):
    wid = lax.axis_index("s") * _NC + lax.axis_index("c")
    base = wid * _BPW

    idx_cps = [
        pltpu.async_copy(
            cat_hbm.at[pl.ds(i * _B + base, _BPW)], idx_v.at[i], sem_i
        )
        for i in range(_NCAT)
    ]
    for cp in idx_cps:
        cp.wait()

    def addoff(t, c):
        i = t // (_BPW // _L)
        k = t % (_BPW // _L)
        v = idx_v[i, pl.ds(k * _L, _L)]
        gbase = (i >> 2) * (4 * _VP) + (i & 3)
        idx_v[i, pl.ds(k * _L, _L)] = gbase + (v << 2)
        return c

    lax.fori_loop(0, _NCAT * (_BPW // _L), addoff, 0)

    def sub(s, c):
        sbase = base + s * _R
        n_cp = pltpu.async_copy(
            o_hbm.at[pl.ds(sbase, _R), pl.ds(0, _L)], ni_v, sem_n
        )
        g_cps = [
            pltpu.async_copy(
                tab_hbm.at[idx_v.at[i, pl.ds(s * _R, _R)]], stage.at[i], sem_g
            )
            for i in range(_NCAT)
        ]
        w_drain = pltpu.make_async_copy(buf, out_hbm.at[pl.ds(sbase, _R)], sem_w)

        @pl.when(s > 0)
        def _():
            w_drain.wait()

        n_cp.wait()

        def conv(r4, c2):
            for dr in range(4):
                r = r4 * 4 + dr
                buf[r, pl.ds(0, _L)] = (
                    ni_v[r].astype(jnp.float32) * (1.0 / _VOCAB)
                )
            return c2

        lax.fori_loop(0, _R // 4, conv, 0)

        for i in range(_NCAT):
            g_cps[i].wait()
            col = _NUM + _D * i

            def rl(r4, c2, i=i, col=col):
                for dr in range(4):
                    r = r4 * 4 + dr
                    buf[r, pl.ds(col, _L)] = stage[i, r, pl.ds(0, _L)]
                    buf[r, pl.ds(col + _L, _L)] = stage[i, r, pl.ds(_L, _L)]
                return c2

            lax.fori_loop(0, _R // 4, rl, 0)

        pltpu.async_copy(buf, out_hbm.at[pl.ds(sbase, _R)], sem_w)
        return c

    lax.fori_loop(0, _NSUB, sub, 0)
    pltpu.make_async_copy(buf, out_hbm.at[pl.ds(base, _R)], sem_w).wait()


_VP = 102400
_VB = 4096
_NVB = _VP // _VB
_NG = 7


def _detile_body(in_ref, out_ref):
    x = in_ref[...]
    x128 = x.reshape(4 * _D, _VB)
    out_ref[...] = x128.T


_tc_detile = pl.pallas_call(
    _detile_body,
    grid=(_NG, _NVB),
    in_specs=[pl.BlockSpec((4, _D, _VB), lambda g, b: (g, 0, b))],
    out_specs=pl.BlockSpec((_VB, 128), lambda g, b: (g * _NVB + b, 0)),
    out_shape=jax.ShapeDtypeStruct((_NG * _VP, 128), jnp.float32),
)


def kernel(O, tables):
    cat_flat = O[:, _NUM:].T.reshape(_NCAT * _B)
    tab128 = _tc_detile(tables.transpose(0, 2, 1))
    tab_flat = tab128.reshape(_NG * _VP * 4, _D)
    return _sc_embed(O, cat_flat, tab_flat)


# speedup vs baseline: 2.9681x; 1.0047x over previous
import functools

import jax
import jax.numpy as jnp
from jax import lax
from jax.experimental import pallas as pl
from jax.experimental.pallas import tpu as pltpu
from jax.experimental.pallas import tpu_sc as plsc

_NUM = 13
_NCAT = 26
_VOCAB = 100000
_D = 32
_B = 16384
_F = _NUM + _NCAT * _D

_NC, _NS, _L = 2, 16, 16
_NW = _NC * _NS
_BPW = _B // _NW
_R = 64
_NSUB = _BPW // _R

_mesh = plsc.VectorSubcoreMesh(
    core_axis_name="c", subcore_axis_name="s", num_cores=_NC, num_subcores=_NS
)


@functools.partial(
    pl.kernel,
    out_type=jax.ShapeDtypeStruct((_B, _F), jnp.float32),
    mesh=_mesh,
    scratch_types=[
        pltpu.VMEM((_NCAT, _BPW), jnp.int32),
        pltpu.VMEM((_NCAT, _R, _D), jnp.float32),
        pltpu.VMEM((_R, _F), jnp.float32),
        pltpu.VMEM((_R, _L), jnp.int32),
        pltpu.SemaphoreType.DMA,
        pltpu.SemaphoreType.DMA,
        pltpu.SemaphoreType.DMA,
        pltpu.SemaphoreType.DMA,
    ],
    compiler_params=pltpu.CompilerParams(use_tc_tiling_on_sc=False),
)
def _sc_embed(o_hbm, cat_hbm, tab_hbm, out_hbm, idx_v, stage, buf, ni_v,
              sem_i, sem_n, sem_g, sem_w):
    wid = lax.axis_index("s") * _NC + lax.axis_index("c")
    base = wid * _BPW

    idx_cps = [
        pltpu.async_copy(
            cat_hbm.at[pl.ds(i * _B + base, _BPW)], idx_v.at[i], sem_i
        )
        for i in range(_NCAT)
    ]

    for i in range(_NCAT):
        idx_cps[i].wait()

        def addoff(k, c, i=i):
            v = idx_v[i, pl.ds(k * _L, _L)]
            gbase = (i >> 2) * (4 * _VP) + (i & 3)
            idx_v[i, pl.ds(k * _L, _L)] = gbase + (v << 2)
            return c

        lax.fori_loop(0, _BPW // _L, addoff, 0)

    def fire_gathers(s):
        for i in range(_NCAT):
            pltpu.async_copy(
                tab_hbm.at[idx_v.at[i, pl.ds(s * _R, _R)]], stage.at[i], sem_g
            )

    fire_gathers(0)

    def sub(s, c):
        sbase = base + s * _R
        n_cp = pltpu.async_copy(
            o_hbm.at[pl.ds(sbase, _R), pl.ds(0, _L)], ni_v, sem_n
        )
        w_drain = pltpu.make_async_copy(buf, out_hbm.at[pl.ds(sbase, _R)], sem_w)

        @pl.when(s > 0)
        def _():
            w_drain.wait()

        n_cp.wait()

        def conv(r4, c2):
            for dr in range(4):
                r = r4 * 4 + dr
                buf[r, pl.ds(0, _L)] = (
                    ni_v[r].astype(jnp.float32) * (1.0 / _VOCAB)
                )
            return c2

        lax.fori_loop(0, _R // 4, conv, 0)

        for i in range(_NCAT):
            pltpu.make_async_copy(
                tab_hbm.at[idx_v.at[i, pl.ds(s * _R, _R)]], stage.at[i], sem_g
            ).wait()
            col = _NUM + _D * i

            def rl(r4, c2, i=i, col=col):
                for dr in range(4):
                    r = r4 * 4 + dr
                    buf[r, pl.ds(col, _L)] = stage[i, r, pl.ds(0, _L)]
                    buf[r, pl.ds(col + _L, _L)] = stage[i, r, pl.ds(_L, _L)]
                return c2

            lax.fori_loop(0, _R // 4, rl, 0)

            @pl.when(s < _NSUB - 1)
            def _(i=i):
                pltpu.async_copy(
                    tab_hbm.at[idx_v.at[i, pl.ds((s + 1) * _R, _R)]],
                    stage.at[i],
                    sem_g,
                )

        pltpu.async_copy(buf, out_hbm.at[pl.ds(sbase, _R)], sem_w)
        return c

    lax.fori_loop(0, _NSUB, sub, 0)
    pltpu.make_async_copy(buf, out_hbm.at[pl.ds(base, _R)], sem_w).wait()


_VP = 102400
_VB = 4096
_NVB = _VP // _VB
_NG = 7


def _detile_body(in_ref, out_ref):
    x = in_ref[...]
    x128 = x.reshape(4 * _D, _VB)
    out_ref[...] = x128.T


_tc_detile = pl.pallas_call(
    _detile_body,
    grid=(_NG, _NVB),
    in_specs=[pl.BlockSpec((4, _D, _VB), lambda g, b: (g, 0, b))],
    out_specs=pl.BlockSpec((_VB, 128), lambda g, b: (g * _NVB + b, 0)),
    out_shape=jax.ShapeDtypeStruct((_NG * _VP, 128), jnp.float32),
)


def kernel(O, tables):
    cat_flat = O[:, _NUM:].T.reshape(_NCAT * _B)
    tab128 = _tc_detile(tables.transpose(0, 2, 1))
    tab_flat = tab128.reshape(_NG * _VP * 4, _D)
    return _sc_embed(O, cat_flat, tab_flat)


# speedup vs baseline: 3.1456x; 1.0598x over previous
import functools

import jax
import jax.numpy as jnp
from jax import lax
from jax.experimental import pallas as pl
from jax.experimental.pallas import tpu as pltpu
from jax.experimental.pallas import tpu_sc as plsc

_NUM = 13
_NCAT = 26
_VOCAB = 100000
_D = 32
_B = 16384
_F = _NUM + _NCAT * _D

_NC, _NS, _L = 2, 16, 16
_NW = _NC * _NS
_BPW = _B // _NW
_R = 64
_NSUB = _BPW // _R

_mesh = plsc.VectorSubcoreMesh(
    core_axis_name="c", subcore_axis_name="s", num_cores=_NC, num_subcores=_NS
)


@functools.partial(
    pl.kernel,
    out_type=jax.ShapeDtypeStruct((_B, _F), jnp.float32),
    mesh=_mesh,
    scratch_types=[
        pltpu.VMEM((_NCAT, _BPW), jnp.int32),
        pltpu.VMEM((_NCAT, _R, _D), jnp.float32),
        pltpu.VMEM((_R, _F), jnp.float32),
        pltpu.VMEM((_R, _L), jnp.int32),
        pltpu.SemaphoreType.DMA,
        pltpu.SemaphoreType.DMA,
        pltpu.SemaphoreType.DMA,
        pltpu.SemaphoreType.DMA,
    ],
    compiler_params=pltpu.CompilerParams(use_tc_tiling_on_sc=False),
)
def _sc_embed(o_hbm, cat_hbm, tab_hbm, out_hbm, idx_v, stage, buf, ni_v,
              sem_i, sem_n, sem_g, sem_w):
    wid = lax.axis_index("s") * _NC + lax.axis_index("c")
    base = wid * _BPW

    idx_cps = [
        pltpu.async_copy(
            cat_hbm.at[pl.ds(i * _B + base, _BPW)], idx_v.at[i], sem_i
        )
        for i in range(_NCAT)
    ]

    for i in range(_NCAT):
        idx_cps[i].wait()

        def addoff(k, c, i=i):
            v = idx_v[i, pl.ds(k * _L, _L)]
            gbase = (i >> 2) * (4 * _VP) + (i & 3)
            idx_v[i, pl.ds(k * _L, _L)] = gbase + (v << 2)
            return c

        lax.fori_loop(0, _BPW // _L, addoff, 0)

    def fire_gathers(s):
        for i in range(_NCAT):
            pltpu.async_copy(
                tab_hbm.at[idx_v.at[i, pl.ds(s * _R, _R)]], stage.at[i], sem_g
            )

    fire_gathers(0)

    def sub(s, c):
        sbase = base + s * _R
        n_cp = pltpu.async_copy(
            o_hbm.at[pl.ds(sbase, _R), pl.ds(0, _L)], ni_v, sem_n
        )
        w_drain = pltpu.make_async_copy(buf, out_hbm.at[pl.ds(sbase, _R)], sem_w)

        @pl.when(s > 0)
        def _():
            w_drain.wait()

        n_cp.wait()

        def conv(r8, c2):
            for dr in range(8):
                r = r8 * 8 + dr
                buf[r, pl.ds(0, _L)] = (
                    ni_v[r].astype(jnp.float32) * (1.0 / _VOCAB)
                )
            return c2

        lax.fori_loop(0, _R // 8, conv, 0)

        for i in range(_NCAT):
            pltpu.make_async_copy(
                tab_hbm.at[idx_v.at[i, pl.ds(s * _R, _R)]], stage.at[i], sem_g
            ).wait()
            col = _NUM + _D * i

            def rl(r8, c2, i=i, col=col):
                for dr in range(8):
                    r = r8 * 8 + dr
                    buf[r, pl.ds(col, _L)] = stage[i, r, pl.ds(0, _L)]
                    buf[r, pl.ds(col + _L, _L)] = stage[i, r, pl.ds(_L, _L)]
                return c2

            lax.fori_loop(0, _R // 8, rl, 0)

            @pl.when(s < _NSUB - 1)
            def _(i=i):
                pltpu.async_copy(
                    tab_hbm.at[idx_v.at[i, pl.ds((s + 1) * _R, _R)]],
                    stage.at[i],
                    sem_g,
                )

        pltpu.async_copy(buf, out_hbm.at[pl.ds(sbase, _R)], sem_w)
        return c

    lax.fori_loop(0, _NSUB, sub, 0)
    pltpu.make_async_copy(buf, out_hbm.at[pl.ds(base, _R)], sem_w).wait()


_VP = 102400
_VB = 4096
_NVB = _VP // _VB
_NG = 7


def _detile_body(in_ref, out_ref):
    x = in_ref[...]
    x256 = x.reshape(8 * _D, _VB)
    y = x256.T
    out_ref[0] = y[:, :128]
    out_ref[1] = y[:, 128:]


_tc_detile = pl.pallas_call(
    _detile_body,
    grid=(_NG // 2 + 1, _NVB),
    in_specs=[pl.BlockSpec((8, _D, _VB), lambda g2, b: (g2, 0, b))],
    out_specs=pl.BlockSpec((2, _VB, 128), lambda g2, b: (g2, b, 0)),
    out_shape=jax.ShapeDtypeStruct((_NG, _VP, 128), jnp.float32),
)


def kernel(O, tables):
    cat_flat = O[:, _NUM:].T.reshape(_NCAT * _B)
    tab3d = _tc_detile(tables.transpose(0, 2, 1))
    tab_flat = tab3d.reshape(_NG * _VP * 4, _D)
    return _sc_embed(O, cat_flat, tab_flat)
